# baseline (speedup 1.0000x reference)
import functools

import jax
import jax.numpy as jnp
from jax import lax
from jax.experimental import pallas as pl
from jax.experimental.pallas import tpu as pltpu
from jax.experimental.pallas import tpu_sc as plsc

_XOR_MASK = 1 << 21
_N_WORKERS = 32
_CH = 16384
_NBUF = 4


def _make_fault_injector(n_words: int, n_faults: int):
    mesh = plsc.VectorSubcoreMesh(core_axis_name="c", subcore_axis_name="s")
    w_per = n_words // _N_WORKERS
    n_chunks = w_per // _CH
    n_vecs = n_faults // 16
    assert n_words % _N_WORKERS == 0 and w_per % _CH == 0
    assert n_faults % 16 == 0

    @functools.partial(
        pl.kernel,
        mesh=mesh,
        out_type=jax.ShapeDtypeStruct((n_words,), jnp.float32),
        scratch_types=[
            pltpu.VMEM((n_faults,), jnp.int32),
            pltpu.VMEM((n_faults + 16,), jnp.int32),
            pltpu.VMEM((n_faults + 16,), jnp.float32),
        ] + [pltpu.VMEM((_CH,), jnp.float32) for _ in range(_NBUF)]
          + [pltpu.SemaphoreType.DMA for _ in range(2 * _NBUF)],
        compiler_params=pltpu.CompilerParams(needs_layout_passes=False),
    )
    def injector(bits_in, idx_hbm, bits_out, idx_f, widx, wval, *rest):
        bufs = rest[:_NBUF]
        isems = rest[_NBUF:2 * _NBUF]
        osems = rest[2 * _NBUF:]
        cid = lax.axis_index("c")
        sid = lax.axis_index("s")
        wid = sid * 2 + cid
        lo_w = wid * w_per
        lanes = lax.iota(jnp.int32, 16)

        for b in range(_NBUF):
            sl = pl.ds(lo_w + b * _CH, _CH)
            pltpu.async_copy(bits_in.at[sl], bufs[b], isems[b])

        pltpu.sync_copy(idx_hbm, idx_f)

        def filt(t, cnt):
            iv = idx_f[pl.ds(t * 16, 16)]
            ov = ((iv & ~0x1F80) | ((iv & 0x1C00) >> 3)
                  | ((iv & 0x380) << 3))
            m = (ov >= lo_w) & (ov < lo_w + w_per)
            plsc.store_compressed(widx.at[pl.ds(cnt, 16)], ov - lo_w, mask=m)
            pop = plsc.all_reduce_population_count(m)
            return cnt + lax.reduce_max(pop, (0,))

        cnt = lax.fori_loop(0, n_vecs, filt, jnp.int32(0))
        nv = (cnt + 15) >> 4

        def apply_faults(b, c):
            lo_c = c * _CH

            def ph_gather(t, _):
                base = t * 16
                wl = widx[pl.ds(base, 16)]
                m = ((base + lanes) < cnt) & (wl >= lo_c) & (wl < lo_c + _CH)
                loc = jnp.where(m, wl - lo_c, 0)
                g = plsc.load_gather(bufs[b], [loc], mask=m)
                gb = plsc.bitcast(g, jnp.int32) ^ _XOR_MASK
                wval[pl.ds(base, 16)] = plsc.bitcast(gb, jnp.float32)
                return 0

            def ph_scatter(t, _):
                base = t * 16
                wl = widx[pl.ds(base, 16)]
                m = ((base + lanes) < cnt) & (wl >= lo_c) & (wl < lo_c + _CH)
                loc = jnp.where(m, wl - lo_c, 0)
                v = wval[pl.ds(base, 16)]
                plsc.store_scatter(bufs[b], [loc], v, mask=m)
                return 0

            lax.fori_loop(0, nv, ph_gather, 0)
            lax.fori_loop(0, nv, ph_scatter, 0)

        def pair_body(g, _):
            c0 = g * _NBUF
            for b in range(_NBUF):
                c = c0 + b
                pltpu.make_async_copy(
                    bits_in.at[pl.ds(0, _CH)], bufs[b], isems[b]).wait()
                apply_faults(b, c)
                pltpu.async_copy(
                    bufs[b], bits_out.at[pl.ds(lo_w + c * _CH, _CH)], osems[b])

            @pl.when(g < n_chunks // _NBUF - 1)
            def _():
                for b in range(_NBUF):
                    c = c0 + b
                    pltpu.make_async_copy(
                        bufs[b], bits_out.at[pl.ds(0, _CH)], osems[b]).wait()
                    sl = pl.ds(lo_w + (c + _NBUF) * _CH, _CH)
                    pltpu.async_copy(bits_in.at[sl], bufs[b], isems[b])

            return 0

        lax.fori_loop(0, n_chunks // _NBUF, pair_body, 0)
        for b in range(_NBUF):
            pltpu.make_async_copy(
                bufs[b], bits_out.at[pl.ds(0, _CH)], osems[b]).wait()

    return injector


def _mm_body(x_ref, w_ref, b_ref, o_ref):
    o_ref[...] = (
        jnp.dot(x_ref[...].astype(jnp.bfloat16),
                w_ref[...].astype(jnp.bfloat16),
                preferred_element_type=jnp.float32)
        + b_ref[...]
    )


def _matmul(bits2d, w, b2d, bm: int):
    m, k = bits2d.shape
    n = w.shape[1]
    return pl.pallas_call(
        _mm_body,
        grid=(m // bm,),
        in_specs=[
            pl.BlockSpec((bm, k), lambda i: (i, 0)),
            pl.BlockSpec((k, n), lambda i: (0, 0)),
            pl.BlockSpec((1, n), lambda i: (0, 0)),
        ],
        out_specs=pl.BlockSpec((bm, n), lambda i: (i, 0)),
        out_shape=jax.ShapeDtypeStruct((m, n), jnp.float32),
        compiler_params=pltpu.CompilerParams(
            dimension_semantics=("parallel",),
        ),
    )(bits2d, w, b2d)


def kernel(input, fault_idx, W, b):
    m, k = input.shape
    assert (m, k) == (8192, 1024)
    x_t = jnp.transpose(
        input.reshape(m // 8, 8, k // 128, 128), (0, 2, 1, 3)).reshape(-1)
    injector = _make_fault_injector(x_t.shape[0], fault_idx.shape[0])
    faulty_t = injector(x_t, fault_idx)
    faulty = jnp.transpose(
        faulty_t.reshape(m // 8, k // 128, 8, 128),
        (0, 2, 1, 3)).reshape(m, k)
    return _matmul(faulty, W, b.reshape(1, -1), bm=512)


# speedup vs baseline: 1.0433x; 1.0433x over previous
import functools

import jax
import jax.numpy as jnp
from jax import lax
from jax.experimental import pallas as pl
from jax.experimental.pallas import tpu as pltpu
from jax.experimental.pallas import tpu_sc as plsc

_XOR_MASK = 1 << 21
_N_WORKERS = 32
_CH = 32768
_NBUF = 2


def _make_fault_injector(n_total: int, n_words: int, n_faults: int, base: int):
    mesh = plsc.VectorSubcoreMesh(core_axis_name="c", subcore_axis_name="s")
    w_per = n_words // _N_WORKERS
    n_chunks = w_per // _CH
    n_vecs = n_faults // 16
    assert n_words % _N_WORKERS == 0 and w_per % _CH == 0
    assert n_faults % 16 == 0

    @functools.partial(
        pl.kernel,
        mesh=mesh,
        out_type=jax.ShapeDtypeStruct((n_words,), jnp.float32),
        scratch_types=[
            pltpu.VMEM((n_faults,), jnp.int32),
            pltpu.VMEM((n_faults + 16,), jnp.int32),
            pltpu.VMEM((n_faults + 16,), jnp.float32),
        ] + [pltpu.VMEM((_CH,), jnp.float32) for _ in range(_NBUF)]
          + [pltpu.SemaphoreType.DMA for _ in range(2 * _NBUF)],
        compiler_params=pltpu.CompilerParams(needs_layout_passes=False),
    )
    def injector(bits_in, idx_hbm, bits_out, idx_f, widx, wval, *rest):
        bufs = rest[:_NBUF]
        isems = rest[_NBUF:2 * _NBUF]
        osems = rest[2 * _NBUF:]
        cid = lax.axis_index("c")
        sid = lax.axis_index("s")
        wid = sid * 2 + cid
        lo_w = wid * w_per
        lanes = lax.iota(jnp.int32, 16)

        for b in range(_NBUF):
            sl = pl.ds(base + lo_w + b * _CH, _CH)
            pltpu.async_copy(bits_in.at[sl], bufs[b], isems[b])

        pltpu.sync_copy(idx_hbm, idx_f)

        def filt(t, cnt):
            iv = idx_f[pl.ds(t * 16, 16)]
            ov = ((iv & ~0x1F80) | ((iv & 0x1C00) >> 3)
                  | ((iv & 0x380) << 3))
            ovr = ov - base
            m = (ovr >= lo_w) & (ovr < lo_w + w_per)
            plsc.store_compressed(widx.at[pl.ds(cnt, 16)], ovr - lo_w, mask=m)
            pop = plsc.all_reduce_population_count(m)
            return cnt + lax.reduce_max(pop, (0,))

        cnt = lax.fori_loop(0, n_vecs, filt, jnp.int32(0))
        nv = (cnt + 15) >> 4

        def apply_faults(b, c):
            lo_c = c * _CH

            def ph_gather(t, _):
                base = t * 16
                wl = widx[pl.ds(base, 16)]
                m = ((base + lanes) < cnt) & (wl >= lo_c) & (wl < lo_c + _CH)
                loc = jnp.where(m, wl - lo_c, 0)
                g = plsc.load_gather(bufs[b], [loc], mask=m)
                gb = plsc.bitcast(g, jnp.int32) ^ _XOR_MASK
                wval[pl.ds(base, 16)] = plsc.bitcast(gb, jnp.float32)
                return 0

            def ph_scatter(t, _):
                base = t * 16
                wl = widx[pl.ds(base, 16)]
                m = ((base + lanes) < cnt) & (wl >= lo_c) & (wl < lo_c + _CH)
                loc = jnp.where(m, wl - lo_c, 0)
                v = wval[pl.ds(base, 16)]
                plsc.store_scatter(bufs[b], [loc], v, mask=m)
                return 0

            lax.fori_loop(0, nv, ph_gather, 0)
            lax.fori_loop(0, nv, ph_scatter, 0)

        def pair_body(g, _):
            c0 = g * _NBUF
            for b in range(_NBUF):
                c = c0 + b
                pltpu.make_async_copy(
                    bits_in.at[pl.ds(0, _CH)], bufs[b], isems[b]).wait()
                apply_faults(b, c)
                pltpu.async_copy(
                    bufs[b], bits_out.at[pl.ds(lo_w + c * _CH, _CH)], osems[b])

            @pl.when(g < n_chunks // _NBUF - 1)
            def _():
                for b in range(_NBUF):
                    c = c0 + b
                    pltpu.make_async_copy(
                        bufs[b], bits_out.at[pl.ds(0, _CH)], osems[b]).wait()
                    sl = pl.ds(base + lo_w + (c + _NBUF) * _CH, _CH)
                    pltpu.async_copy(bits_in.at[sl], bufs[b], isems[b])

            return 0

        lax.fori_loop(0, n_chunks // _NBUF, pair_body, 0)
        for b in range(_NBUF):
            pltpu.make_async_copy(
                bufs[b], bits_out.at[pl.ds(0, _CH)], osems[b]).wait()

    return injector


def _mm_body(x_ref, w_ref, b_ref, o_ref):
    o_ref[...] = (
        jnp.dot(x_ref[...].astype(jnp.bfloat16),
                w_ref[...].astype(jnp.bfloat16),
                preferred_element_type=jnp.float32)
        + b_ref[...]
    )


def _mm_body_alias(x_ref, w_ref, b_ref, carrier_ref, o_ref):
    _mm_body(x_ref, w_ref, b_ref, o_ref)


def _matmul_into(x2d, w, b2d, carrier, out_rows, row_block_off: int, bm: int):
    m, k = x2d.shape
    n = w.shape[1]
    in_specs = [
        pl.BlockSpec((bm, k), lambda i: (i, 0)),
        pl.BlockSpec((k, n), lambda i: (0, 0)),
        pl.BlockSpec((1, n), lambda i: (0, 0)),
    ]
    args = [x2d, w, b2d]
    body = _mm_body
    aliases = {}
    if carrier is not None:
        in_specs.append(pl.BlockSpec(memory_space=pl.ANY))
        args.append(carrier)
        body = _mm_body_alias
        aliases = {3: 0}
    return pl.pallas_call(
        body,
        grid=(m // bm,),
        in_specs=in_specs,
        out_specs=pl.BlockSpec(
            (bm, n), lambda i, _o=row_block_off: (i + _o, 0)),
        out_shape=jax.ShapeDtypeStruct((out_rows, n), jnp.float32),
        input_output_aliases=aliases,
        compiler_params=pltpu.CompilerParams(
            dimension_semantics=("arbitrary",),
        ),
    )(*args)


def kernel(input, fault_idx, W, b):
    m, k = input.shape
    assert (m, k) == (8192, 1024)
    x_t = jnp.transpose(
        input.reshape(m // 8, 8, k // 128, 128), (0, 2, 1, 3)).reshape(-1)
    n_total = x_t.shape[0]
    half = n_total // 2
    b2d = b.reshape(1, -1)
    bm = 1024

    def faulty_rows(seg_t, rows):
        return jnp.transpose(
            seg_t.reshape(rows // 8, k // 128, 8, 128),
            (0, 2, 1, 3)).reshape(rows, k)

    inj_a = _make_fault_injector(n_total, half, fault_idx.shape[0], 0)
    inj_b = _make_fault_injector(n_total, half, fault_idx.shape[0], half)
    fa = faulty_rows(inj_a(x_t, fault_idx), m // 2)
    fb = faulty_rows(inj_b(x_t, fault_idx), m // 2)
    y = _matmul_into(fa, W, b2d, None, m, 0, bm)
    return _matmul_into(fb, W, b2d, y, m, (m // 2) // bm, bm)


# speedup vs baseline: 1.0766x; 1.0319x over previous
import functools

import jax
import jax.numpy as jnp
from jax import lax
from jax.experimental import pallas as pl
from jax.experimental.pallas import tpu as pltpu
from jax.experimental.pallas import tpu_sc as plsc

_XOR_MASK = 1 << 21
_N_WORKERS = 32
_CH = 32768
_NBUF = 2


def _make_fault_injector(n_words: int, n_faults: int):
    mesh = plsc.VectorSubcoreMesh(core_axis_name="c", subcore_axis_name="s")
    w_per = n_words // _N_WORKERS
    n_chunks = w_per // _CH
    n_vecs = n_faults // 16
    assert n_words % _N_WORKERS == 0 and w_per % (_CH * _NBUF) == 0
    assert n_faults % 16 == 0

    @functools.partial(
        pl.kernel,
        mesh=mesh,
        out_type=jax.ShapeDtypeStruct((n_words,), jnp.float32),
        scratch_types=[
            pltpu.VMEM((n_faults,), jnp.int32),
            pltpu.VMEM((n_faults + 16,), jnp.int32),
            pltpu.VMEM((n_faults + 16,), jnp.float32),
        ] + [pltpu.VMEM((_CH,), jnp.float32) for _ in range(_NBUF)]
          + [pltpu.SemaphoreType.DMA for _ in range(2 * _NBUF)],
        compiler_params=pltpu.CompilerParams(needs_layout_passes=False),
    )
    def injector(bits_in, idx_hbm, bits_out, idx_f, widx, wval, *rest):
        bufs = rest[:_NBUF]
        isems = rest[_NBUF:2 * _NBUF]
        osems = rest[2 * _NBUF:]
        cid = lax.axis_index("c")
        sid = lax.axis_index("s")
        wid = sid * 2 + cid
        lo_w = wid * w_per
        lanes = lax.iota(jnp.int32, 16)

        for b in range(_NBUF):
            sl = pl.ds(lo_w + b * _CH, _CH)
            pltpu.async_copy(bits_in.at[sl], bufs[b], isems[b])

        pltpu.sync_copy(idx_hbm, idx_f)

        def filt(t, cnt):
            iv = idx_f[pl.ds(t * 16, 16)]
            ov = ((iv & ~0x1F80) | ((iv & 0x1C00) >> 3)
                  | ((iv & 0x380) << 3))
            m = (ov >= lo_w) & (ov < lo_w + w_per)
            plsc.store_compressed(widx.at[pl.ds(cnt, 16)], ov - lo_w, mask=m)
            pop = plsc.all_reduce_population_count(m)
            return cnt + lax.reduce_max(pop, (0,))

        cnt = lax.fori_loop(0, n_vecs, filt, jnp.int32(0))
        nv = (cnt + 15) >> 4

        def apply_faults(b, c):
            lo_c = c * _CH

            def ph_gather(t, _):
                base = t * 16
                wl = widx[pl.ds(base, 16)]
                m = ((base + lanes) < cnt) & (wl >= lo_c) & (wl < lo_c + _CH)
                loc = jnp.where(m, wl - lo_c, 0)
                g = plsc.load_gather(bufs[b], [loc], mask=m)
                gb = plsc.bitcast(g, jnp.int32) ^ _XOR_MASK
                wval[pl.ds(base, 16)] = plsc.bitcast(gb, jnp.float32)
                return 0

            def ph_scatter(t, _):
                base = t * 16
                wl = widx[pl.ds(base, 16)]
                m = ((base + lanes) < cnt) & (wl >= lo_c) & (wl < lo_c + _CH)
                loc = jnp.where(m, wl - lo_c, 0)
                v = wval[pl.ds(base, 16)]
                plsc.store_scatter(bufs[b], [loc], v, mask=m)
                return 0

            lax.fori_loop(0, nv, ph_gather, 0)
            lax.fori_loop(0, nv, ph_scatter, 0)

        def ring_body(g, _):
            c0 = g * _NBUF
            for b in range(_NBUF):
                c = c0 + b
                pltpu.make_async_copy(
                    bits_in.at[pl.ds(0, _CH)], bufs[b], isems[b]).wait()
                apply_faults(b, c)
                pltpu.async_copy(
                    bufs[b], bits_out.at[pl.ds(lo_w + c * _CH, _CH)], osems[b])

            @pl.when(g < n_chunks // _NBUF - 1)
            def _():
                for b in range(_NBUF):
                    c = c0 + b
                    pltpu.make_async_copy(
                        bufs[b], bits_out.at[pl.ds(0, _CH)], osems[b]).wait()
                    sl = pl.ds(lo_w + (c + _NBUF) * _CH, _CH)
                    pltpu.async_copy(bits_in.at[sl], bufs[b], isems[b])

            return 0

        lax.fori_loop(0, n_chunks // _NBUF, ring_body, 0)
        for b in range(_NBUF):
            pltpu.make_async_copy(
                bufs[b], bits_out.at[pl.ds(0, _CH)], osems[b]).wait()

    return injector


def _mm_body(x_ref, w_ref, b_ref, o_ref):
    o_ref[...] = (
        jnp.dot(x_ref[...].astype(jnp.bfloat16),
                w_ref[...].astype(jnp.bfloat16),
                preferred_element_type=jnp.float32)
        + b_ref[...]
    )


def _matmul(x2d, w, b2d, bm: int):
    m, k = x2d.shape
    n = w.shape[1]
    return pl.pallas_call(
        _mm_body,
        grid=(m // bm,),
        in_specs=[
            pl.BlockSpec((bm, k), lambda i: (i, 0)),
            pl.BlockSpec((k, n), lambda i: (0, 0)),
            pl.BlockSpec((1, n), lambda i: (0, 0)),
        ],
        out_specs=pl.BlockSpec((bm, n), lambda i: (i, 0)),
        out_shape=jax.ShapeDtypeStruct((m, n), jnp.float32),
        compiler_params=pltpu.CompilerParams(
            dimension_semantics=("parallel",),
        ),
    )(x2d, w, b2d)


def kernel(input, fault_idx, W, b):
    m, k = input.shape
    assert (m, k) == (8192, 1024)
    x_t = jnp.transpose(
        input.reshape(m // 8, 8, k // 128, 128), (0, 2, 1, 3)).reshape(-1)
    injector = _make_fault_injector(x_t.shape[0], fault_idx.shape[0])
    faulty_t = injector(x_t, fault_idx)
    faulty = jnp.transpose(
        faulty_t.reshape(m // 8, k // 128, 8, 128),
        (0, 2, 1, 3)).reshape(m, k)
    return _matmul(faulty, W, b.reshape(1, -1), bm=1024)
